# baseline (speedup 1.0000x reference)
import functools

import jax
import jax.numpy as jnp
from jax import lax
from jax.experimental import pallas as pl
from jax.experimental.pallas import tpu as pltpu
from jax.experimental.pallas import tpu_sc as plsc

N = 10000
E = 320000
K = 64
D = 32

NC, NS = 2, 16
NW = NC * NS

N_PAD = 10240
E_PAD = 327680
A_PAD = N_PAD * K

IDX = 512

_MESH = plsc.VectorSubcoreMesh(
    core_axis_name="c", subcore_axis_name="s", num_cores=NC, num_subcores=NS)


def _worker_id():
    return lax.axis_index("s") * NC + lax.axis_index("c")


_CB = 512
_NCHUNK_B = E_PAD // NW // _CB


@functools.partial(
    pl.kernel,
    out_type=jax.ShapeDtypeStruct((E_PAD, D), jnp.float32),
    mesh=_MESH,
    compiler_params=pltpu.CompilerParams(use_tc_tiling_on_sc=False),
    scratch_types=[
        pltpu.VMEM((_CB,), jnp.int32),
        pltpu.VMEM((_CB,), jnp.int32),
        pltpu.VMEM((_CB,), jnp.float32),
        pltpu.VMEM((_CB, D), jnp.float32),
        pltpu.VMEM((_CB, D), jnp.float32),
        pltpu.VMEM((_CB, D), jnp.float32),
        pltpu.SemaphoreType.DMA,
    ],
)
def _edge_messages(u_hbm, v_hbm, src_hbm, dst_hbm, sp_hbm, out_hbm,
                   src_v, dst_v, sp_v, urow_v, vrow_v, out_v, sem):
    ebase = _worker_id() * (E_PAD // NW)

    def chunk_body(ci, carry):
        base = ebase + ci * _CB
        pltpu.sync_copy(src_hbm.at[pl.ds(base, _CB)], src_v)
        pltpu.sync_copy(dst_hbm.at[pl.ds(base, _CB)], dst_v)
        pltpu.sync_copy(sp_hbm.at[pl.ds(base, _CB)], sp_v)
        descs = []
        for j in range(_CB // IDX):
            sl = pl.ds(j * IDX, IDX)
            descs.append(pltpu.async_copy(
                u_hbm.at[src_v.at[sl]], urow_v.at[sl], sem))
            descs.append(pltpu.async_copy(
                v_hbm.at[dst_v.at[sl]], vrow_v.at[sl], sem))
        for d_ in descs:
            d_.wait()

        def group_body(g, c):
            spv = sp_v[pl.ds(g * 16, 16)]
            for i in range(16):
                e = g * 16 + i
                s = spv[i]
                for h in range(D // 16):
                    hs = pl.ds(h * 16, 16)
                    out_v[e, hs] = jnp.maximum(
                        vrow_v[e, hs] + urow_v[e, hs] * s, 0.0)
            return c

        lax.fori_loop(0, _CB // 16, group_body, 0)
        pltpu.sync_copy(out_v, out_hbm.at[pl.ds(base, _CB)])
        return carry

    lax.fori_loop(0, _NCHUNK_B, chunk_body, 0)


@functools.partial(
    pl.kernel,
    out_type=jax.ShapeDtypeStruct((E_PAD,), jnp.float32),
    mesh=_MESH,
    compiler_params=pltpu.CompilerParams(use_tc_tiling_on_sc=False),
    scratch_types=[
        pltpu.VMEM((_CB,), jnp.int32),
        pltpu.VMEM((_CB,), jnp.int32),
        pltpu.VMEM((_CB,), jnp.float32),
        pltpu.VMEM((_CB, D), jnp.float32),
        pltpu.VMEM((_CB, D), jnp.float32),
        pltpu.VMEM((_CB,), jnp.float32),
        pltpu.VMEM((D,), jnp.float32),
        pltpu.SemaphoreType.DMA,
    ],
)
def _edge_scores(u_hbm, v_hbm, src_hbm, dst_hbm, sp_hbm, wp_hbm, out_hbm,
                 src_v, dst_v, sp_v, urow_v, vrow_v, out_v, wp_v, sem):
    ebase = _worker_id() * (E_PAD // NW)
    pltpu.sync_copy(wp_hbm, wp_v)
    w0 = wp_v[pl.ds(0, 16)]
    w1 = wp_v[pl.ds(16, 16)]
    lane = lax.broadcasted_iota(jnp.int32, (16,), 0)
    perms = [lane ^ b for b in (1, 2, 4, 8)]
    masks = [lane == i for i in range(16)]

    def chunk_body(ci, carry):
        base = ebase + ci * _CB
        pltpu.sync_copy(src_hbm.at[pl.ds(base, _CB)], src_v)
        pltpu.sync_copy(dst_hbm.at[pl.ds(base, _CB)], dst_v)
        pltpu.sync_copy(sp_hbm.at[pl.ds(base, _CB)], sp_v)
        descs = []
        for j in range(_CB // IDX):
            sl = pl.ds(j * IDX, IDX)
            descs.append(pltpu.async_copy(
                u_hbm.at[src_v.at[sl]], urow_v.at[sl], sem))
            descs.append(pltpu.async_copy(
                v_hbm.at[dst_v.at[sl]], vrow_v.at[sl], sem))
        for d_ in descs:
            d_.wait()

        def group_body(g, c):
            spv = sp_v[pl.ds(g * 16, 16)]
            res = jnp.zeros((16,), jnp.float32)
            for i in range(16):
                e = g * 16 + i
                s = spv[i]
                m0 = jnp.maximum(
                    vrow_v[e, pl.ds(0, 16)] + urow_v[e, pl.ds(0, 16)] * s, 0.0)
                m1 = jnp.maximum(
                    vrow_v[e, pl.ds(16, 16)] + urow_v[e, pl.ds(16, 16)] * s,
                    0.0)
                t = m0 * w0 + m1 * w1
                for p in perms:
                    t = t + jnp.take_along_axis(t, p, axis=0)
                res = jnp.where(masks[i], t, res)
            out_v[pl.ds(g * 16, 16)] = res
            return c

        lax.fori_loop(0, _CB // 16, group_body, 0)
        pltpu.sync_copy(out_v, out_hbm.at[pl.ds(base, _CB)])
        return carry

    lax.fori_loop(0, _NCHUNK_B, chunk_body, 0)


_CNODES = 8
_CA = _CNODES * K
_NCHUNK_C = N_PAD // NW // _CNODES


@functools.partial(
    pl.kernel,
    out_type=jax.ShapeDtypeStruct((N_PAD, D), jnp.float32),
    mesh=_MESH,
    compiler_params=pltpu.CompilerParams(use_tc_tiling_on_sc=False),
    scratch_types=[
        pltpu.VMEM((_CA,), jnp.int32),
        pltpu.VMEM((_CA, D), jnp.float32),
        pltpu.VMEM((_CNODES, D), jnp.float32),
        pltpu.SemaphoreType.DMA,
    ],
)
def _anchor_mean(msg_hbm, anchor_hbm, out_hbm, idx_v, rows_v, acc_v, sem):
    nbase = _worker_id() * (N_PAD // NW)

    def chunk_body(ci, carry):
        n0 = nbase + ci * _CNODES
        pltpu.sync_copy(anchor_hbm.at[pl.ds(n0 * K, _CA)], idx_v)
        descs = []
        for j in range(_CA // IDX):
            sl = pl.ds(j * IDX, IDX)
            descs.append(pltpu.async_copy(
                msg_hbm.at[idx_v.at[sl]], rows_v.at[sl], sem))
        for d_ in descs:
            d_.wait()

        def node_body(ni, c):
            def k_body(k, accs):
                a0, a1 = accs
                r = ni * K + k
                a0 = a0 + rows_v[r, pl.ds(0, 16)]
                a1 = a1 + rows_v[r, pl.ds(16, 16)]
                return (a0, a1)

            a0, a1 = lax.fori_loop(
                0, K, k_body,
                (jnp.zeros((16,), jnp.float32), jnp.zeros((16,), jnp.float32)),
                unroll=8)
            acc_v[ni, pl.ds(0, 16)] = a0 * (1.0 / K)
            acc_v[ni, pl.ds(16, 16)] = a1 * (1.0 / K)
            return c

        lax.fori_loop(0, _CNODES, node_body, 0)
        pltpu.sync_copy(acc_v, out_hbm.at[pl.ds(n0, _CNODES)])
        return carry

    lax.fori_loop(0, _NCHUNK_C, chunk_body, 0)


_CF = 2048
_NCHUNK_F = A_PAD // NW // _CF


@functools.partial(
    pl.kernel,
    out_type=jax.ShapeDtypeStruct((A_PAD,), jnp.float32),
    mesh=_MESH,
    compiler_params=pltpu.CompilerParams(use_tc_tiling_on_sc=False),
    scratch_types=[
        pltpu.VMEM((_CF,), jnp.int32),
        pltpu.VMEM((_CF,), jnp.float32),
        pltpu.SemaphoreType.DMA,
    ],
)
def _scalar_gather(p_hbm, anchor_hbm, out_hbm, idx_v, val_v, sem):
    base0 = _worker_id() * (A_PAD // NW)

    def chunk_body(ci, carry):
        base = base0 + ci * _CF
        pltpu.sync_copy(anchor_hbm.at[pl.ds(base, _CF)], idx_v)
        descs = []
        for j in range(_CF // IDX):
            sl = pl.ds(j * IDX, IDX)
            descs.append(pltpu.async_copy(
                p_hbm.at[idx_v.at[sl]], val_v.at[sl], sem))
        for d_ in descs:
            d_.wait()
        pltpu.sync_copy(val_v, out_hbm.at[pl.ds(base, _CF)])
        return carry

    lax.fori_loop(0, _NCHUNK_F, chunk_body, 0)


def _feats_pre(feat, wpre, bpre, wu, bu, wv, bv):
    def body(f_ref, wp_ref, bp_ref, wu_ref, bu_ref, wv_ref, bv_ref,
             u_out, v_out):
        x = jnp.dot(f_ref[...], wp_ref[...],
                    preferred_element_type=jnp.float32) + bp_ref[...]
        u_out[...] = jnp.dot(x, wu_ref[...],
                             preferred_element_type=jnp.float32) + bu_ref[...]
        v_out[...] = jnp.dot(x, wv_ref[...],
                             preferred_element_type=jnp.float32) + bv_ref[...]

    n = feat.shape[0]
    return pl.pallas_call(
        body,
        out_shape=(jax.ShapeDtypeStruct((n, D), jnp.float32),
                   jax.ShapeDtypeStruct((n, D), jnp.float32)),
    )(feat, wpre, bpre, wu, bu, wv, bv)


def _feats(x, wu, bu, wv, bv):
    def body(x_ref, wu_ref, bu_ref, wv_ref, bv_ref, u_out, v_out):
        xv = x_ref[...]
        u_out[...] = jnp.dot(xv, wu_ref[...],
                             preferred_element_type=jnp.float32) + bu_ref[...]
        v_out[...] = jnp.dot(xv, wv_ref[...],
                             preferred_element_type=jnp.float32) + bv_ref[...]

    n = x.shape[0]
    return pl.pallas_call(
        body,
        out_shape=(jax.ShapeDtypeStruct((n, D), jnp.float32),
                   jax.ShapeDtypeStruct((n, D), jnp.float32)),
    )(x, wu, bu, wv, bv)


def _normalize(pos, bp):
    def body(p_ref, bp_ref, o_ref):
        x = p_ref[...] + bp_ref[...]
        ss = jnp.sum(x * x, axis=1, keepdims=True)
        o_ref[...] = x / jnp.maximum(jnp.sqrt(ss), 1e-12)

    return pl.pallas_call(
        body,
        out_shape=jax.ShapeDtypeStruct(pos.shape, jnp.float32),
    )(pos, bp)


def kernel(feat, sp_dist, dists_max, edge_index, anchor_eid,
           W_pre, b_pre, Wu1, bu1, Wv1, bv1, Wp1, bp1,
           Wu2, bu2, Wv2, bv2, Wp2, bp2):
    src = jnp.pad(edge_index[0].astype(jnp.int32), (0, E_PAD - E))
    dst = jnp.pad(edge_index[1].astype(jnp.int32), (0, E_PAD - E))
    sp = jnp.pad(sp_dist[:, 0], (0, E_PAD - E))
    anchor = jnp.pad(anchor_eid.astype(jnp.int32), (0, A_PAD - N * K))

    u1, v1 = _feats_pre(feat, W_pre, b_pre.reshape(1, D),
                        Wu1, bu1.reshape(1, D), Wv1, bv1.reshape(1, D))
    rmsg1 = _edge_messages(u1, v1, src, dst, sp)
    x2 = _anchor_mean(rmsg1, anchor)
    u2, v2 = _feats(x2, Wu2, bu2.reshape(1, D), Wv2, bv2.reshape(1, D))
    p2 = _edge_scores(u2, v2, src, dst, sp, Wp2.reshape(D))
    pos = _scalar_gather(p2, anchor).reshape(N_PAD, K)[:N]
    return _normalize(pos, bp2.reshape(1, 1))


# speedup vs baseline: 1.2568x; 1.2568x over previous
import functools

import jax
import jax.numpy as jnp
from jax import lax
from jax.experimental import pallas as pl
from jax.experimental.pallas import tpu as pltpu
from jax.experimental.pallas import tpu_sc as plsc

N = 10000
E = 320000
K = 64
D = 32

NC, NS = 2, 16
NW = NC * NS

N_PAD = 10240
E_PAD = 327680
A_PAD = N_PAD * K

IDX = 128

_MESH = plsc.VectorSubcoreMesh(
    core_axis_name="c", subcore_axis_name="s", num_cores=NC, num_subcores=NS)

_SC_PARAMS = pltpu.CompilerParams(use_tc_tiling_on_sc=False)


def _worker_id():
    return lax.axis_index("s") * NC + lax.axis_index("c")


_CB = 512
_NCHUNK_B = E_PAD // NW // _CB


def _edge_buffers():
    return [
        pltpu.VMEM((2, _CB), jnp.int32),
        pltpu.VMEM((2, _CB), jnp.int32),
        pltpu.VMEM((2, _CB), jnp.float32),
        pltpu.VMEM((2, _CB, D), jnp.float32),
        pltpu.VMEM((2, _CB, D), jnp.float32),
        pltpu.SemaphoreType.DMA((2,)),
        pltpu.SemaphoreType.DMA((2,)),
    ]


def _edge_pipeline(u_hbm, v_hbm, src_hbm, dst_hbm, sp_hbm,
                   src_v, dst_v, sp_v, urow_v, vrow_v, isem, gsem,
                   compute_chunk):
    ebase = _worker_id() * (E_PAD // NW)

    def fire_idx(ci, b):
        base = ebase + ci * _CB
        pltpu.async_copy(src_hbm.at[pl.ds(base, _CB)], src_v.at[b], isem.at[b])
        pltpu.async_copy(dst_hbm.at[pl.ds(base, _CB)], dst_v.at[b], isem.at[b])
        pltpu.async_copy(sp_hbm.at[pl.ds(base, _CB)], sp_v.at[b], isem.at[b])

    def wait_idx(b):
        for hbm, ref in ((src_hbm, src_v), (dst_hbm, dst_v), (sp_hbm, sp_v)):
            pltpu.make_async_copy(hbm.at[pl.ds(0, _CB)], ref.at[b],
                                  isem.at[b]).wait()

    def fire_gathers(b):
        for j in range(_CB // IDX):
            sl = pl.ds(j * IDX, IDX)
            pltpu.async_copy(u_hbm.at[src_v.at[b].at[sl]],
                             urow_v.at[b].at[sl], gsem.at[b])
            pltpu.async_copy(v_hbm.at[dst_v.at[b].at[sl]],
                             vrow_v.at[b].at[sl], gsem.at[b])

    def wait_gathers(b):
        for j in range(_CB // IDX):
            sl = pl.ds(j * IDX, IDX)
            pltpu.make_async_copy(u_hbm.at[pl.ds(0, IDX)],
                                  urow_v.at[b].at[sl], gsem.at[b]).wait()
            pltpu.make_async_copy(v_hbm.at[pl.ds(0, IDX)],
                                  vrow_v.at[b].at[sl], gsem.at[b]).wait()

    fire_idx(0, 0)
    wait_idx(0)
    fire_gathers(0)
    fire_idx(1, 1)

    def pair_body(jj, carry):
        for b in (0, 1):
            ci = jj * 2 + b
            wait_gathers(b)
            wait_idx(1 - b)
            fire_gathers(1 - b)
            compute_chunk(b, ebase + ci * _CB)
            fire_idx((ci + 2) % _NCHUNK_B, b)
        return carry

    lax.fori_loop(0, _NCHUNK_B // 2, pair_body, 0)
    wait_idx(1)
    wait_gathers(0)


@functools.partial(
    pl.kernel,
    out_type=jax.ShapeDtypeStruct((E_PAD, D), jnp.float32),
    mesh=_MESH,
    compiler_params=_SC_PARAMS,
    scratch_types=_edge_buffers() + [pltpu.VMEM((_CB, D), jnp.float32)],
)
def _edge_messages(u_hbm, v_hbm, src_hbm, dst_hbm, sp_hbm, out_hbm,
                   src_v, dst_v, sp_v, urow_v, vrow_v, isem, gsem, out_v):

    def compute_chunk(b, base):
        def group_body(g, c):
            spv = sp_v[b, pl.ds(g * 16, 16)]
            for i in range(16):
                e = g * 16 + i
                s = spv[i]
                for h in range(D // 16):
                    hs = pl.ds(h * 16, 16)
                    out_v[e, hs] = jnp.maximum(
                        vrow_v[b, e, hs] + urow_v[b, e, hs] * s, 0.0)
            return c

        lax.fori_loop(0, _CB // 16, group_body, 0)
        pltpu.sync_copy(out_v, out_hbm.at[pl.ds(base, _CB)])

    _edge_pipeline(u_hbm, v_hbm, src_hbm, dst_hbm, sp_hbm,
                   src_v, dst_v, sp_v, urow_v, vrow_v, isem, gsem,
                   compute_chunk)


@functools.partial(
    pl.kernel,
    out_type=jax.ShapeDtypeStruct((E_PAD,), jnp.float32),
    mesh=_MESH,
    compiler_params=_SC_PARAMS,
    scratch_types=_edge_buffers() + [pltpu.VMEM((_CB,), jnp.float32),
                                     pltpu.VMEM((D,), jnp.float32)],
)
def _edge_scores(u_hbm, v_hbm, src_hbm, dst_hbm, sp_hbm, wp_hbm, out_hbm,
                 src_v, dst_v, sp_v, urow_v, vrow_v, isem, gsem, out_v, wp_v):
    pltpu.sync_copy(wp_hbm, wp_v)
    w0 = wp_v[pl.ds(0, 16)]
    w1 = wp_v[pl.ds(16, 16)]
    lane = lax.broadcasted_iota(jnp.int32, (16,), 0)
    perms = [lane ^ bb for bb in (1, 2, 4, 8)]
    masks = [lane == i for i in range(16)]

    def compute_chunk(b, base):
        def group_body(g, c):
            spv = sp_v[b, pl.ds(g * 16, 16)]
            res = jnp.zeros((16,), jnp.float32)
            for i in range(16):
                e = g * 16 + i
                s = spv[i]
                m0 = jnp.maximum(
                    vrow_v[b, e, pl.ds(0, 16)]
                    + urow_v[b, e, pl.ds(0, 16)] * s, 0.0)
                m1 = jnp.maximum(
                    vrow_v[b, e, pl.ds(16, 16)]
                    + urow_v[b, e, pl.ds(16, 16)] * s, 0.0)
                t = m0 * w0 + m1 * w1
                for p in perms:
                    t = t + jnp.take_along_axis(t, p, axis=0)
                res = jnp.where(masks[i], t, res)
            out_v[pl.ds(g * 16, 16)] = res
            return c

        lax.fori_loop(0, _CB // 16, group_body, 0)
        pltpu.sync_copy(out_v, out_hbm.at[pl.ds(base, _CB)])

    _edge_pipeline(u_hbm, v_hbm, src_hbm, dst_hbm, sp_hbm,
                   src_v, dst_v, sp_v, urow_v, vrow_v, isem, gsem,
                   compute_chunk)


_CNODES = 8
_CA = _CNODES * K
_NCHUNK_C = N_PAD // NW // _CNODES


@functools.partial(
    pl.kernel,
    out_type=jax.ShapeDtypeStruct((N_PAD, D), jnp.float32),
    mesh=_MESH,
    compiler_params=_SC_PARAMS,
    scratch_types=[
        pltpu.VMEM((2, _CA), jnp.int32),
        pltpu.VMEM((2, _CA, D), jnp.float32),
        pltpu.VMEM((_CNODES, D), jnp.float32),
        pltpu.SemaphoreType.DMA((2,)),
        pltpu.SemaphoreType.DMA((2,)),
    ],
)
def _anchor_mean(msg_hbm, anchor_hbm, out_hbm, idx_v, rows_v, acc_v,
                 isem, gsem):
    nbase = _worker_id() * (N_PAD // NW)

    def fire_idx(ci, b):
        a0 = (nbase + ci * _CNODES) * K
        pltpu.async_copy(anchor_hbm.at[pl.ds(a0, _CA)], idx_v.at[b],
                         isem.at[b])

    def wait_idx(b):
        pltpu.make_async_copy(anchor_hbm.at[pl.ds(0, _CA)], idx_v.at[b],
                              isem.at[b]).wait()

    def fire_gathers(b):
        for j in range(_CA // IDX):
            sl = pl.ds(j * IDX, IDX)
            pltpu.async_copy(msg_hbm.at[idx_v.at[b].at[sl]],
                             rows_v.at[b].at[sl], gsem.at[b])

    def wait_gathers(b):
        for j in range(_CA // IDX):
            sl = pl.ds(j * IDX, IDX)
            pltpu.make_async_copy(msg_hbm.at[pl.ds(0, IDX)],
                                  rows_v.at[b].at[sl], gsem.at[b]).wait()

    fire_idx(0, 0)
    wait_idx(0)
    fire_gathers(0)
    fire_idx(1, 1)

    def pair_body(jj, carry):
        for b in (0, 1):
            ci = jj * 2 + b
            wait_gathers(b)
            wait_idx(1 - b)
            fire_gathers(1 - b)
            fire_idx((ci + 2) % _NCHUNK_C, b)

            def node_body(ni, c):
                def k_body(k, accs):
                    a0, a1 = accs
                    r = ni * K + k
                    a0 = a0 + rows_v[b, r, pl.ds(0, 16)]
                    a1 = a1 + rows_v[b, r, pl.ds(16, 16)]
                    return (a0, a1)

                a0, a1 = lax.fori_loop(
                    0, K, k_body,
                    (jnp.zeros((16,), jnp.float32),
                     jnp.zeros((16,), jnp.float32)),
                    unroll=8)
                acc_v[ni, pl.ds(0, 16)] = a0 * (1.0 / K)
                acc_v[ni, pl.ds(16, 16)] = a1 * (1.0 / K)
                return c

            lax.fori_loop(0, _CNODES, node_body, 0)
            pltpu.sync_copy(acc_v,
                            out_hbm.at[pl.ds(nbase + ci * _CNODES, _CNODES)])
        return carry

    lax.fori_loop(0, _NCHUNK_C // 2, pair_body, 0)
    wait_idx(1)
    wait_gathers(0)


_CF = 2048
_NCHUNK_F = A_PAD // NW // _CF


@functools.partial(
    pl.kernel,
    out_type=jax.ShapeDtypeStruct((A_PAD,), jnp.float32),
    mesh=_MESH,
    compiler_params=_SC_PARAMS,
    scratch_types=[
        pltpu.VMEM((2, _CF), jnp.int32),
        pltpu.VMEM((2, _CF), jnp.float32),
        pltpu.SemaphoreType.DMA((2,)),
        pltpu.SemaphoreType.DMA((2,)),
    ],
)
def _scalar_gather(p_hbm, anchor_hbm, out_hbm, idx_v, val_v, isem, gsem):
    base0 = _worker_id() * (A_PAD // NW)

    def fire_idx(ci, b):
        base = base0 + ci * _CF
        pltpu.async_copy(anchor_hbm.at[pl.ds(base, _CF)], idx_v.at[b],
                         isem.at[b])

    def wait_idx(b):
        pltpu.make_async_copy(anchor_hbm.at[pl.ds(0, _CF)], idx_v.at[b],
                              isem.at[b]).wait()

    def fire_gathers(b):
        for j in range(_CF // IDX):
            sl = pl.ds(j * IDX, IDX)
            pltpu.async_copy(p_hbm.at[idx_v.at[b].at[sl]],
                             val_v.at[b].at[sl], gsem.at[b])

    def wait_gathers(b):
        for j in range(_CF // IDX):
            sl = pl.ds(j * IDX, IDX)
            pltpu.make_async_copy(p_hbm.at[pl.ds(0, IDX)],
                                  val_v.at[b].at[sl], gsem.at[b]).wait()

    fire_idx(0, 0)
    wait_idx(0)
    fire_gathers(0)
    fire_idx(1, 1)

    def pair_body(jj, carry):
        for b in (0, 1):
            ci = jj * 2 + b
            wait_gathers(b)
            wait_idx(1 - b)
            fire_gathers(1 - b)
            fire_idx((ci + 2) % _NCHUNK_F, b)
            pltpu.sync_copy(val_v.at[b],
                            out_hbm.at[pl.ds(base0 + ci * _CF, _CF)])
        return carry

    lax.fori_loop(0, _NCHUNK_F // 2, pair_body, 0)
    wait_idx(1)
    wait_gathers(0)


def _feats_pre(feat, wpre, bpre, wu, bu, wv, bv):
    def body(f_ref, wp_ref, bp_ref, wu_ref, bu_ref, wv_ref, bv_ref,
             u_out, v_out):
        x = jnp.dot(f_ref[...], wp_ref[...],
                    preferred_element_type=jnp.float32) + bp_ref[...]
        u_out[...] = jnp.dot(x, wu_ref[...],
                             preferred_element_type=jnp.float32) + bu_ref[...]
        v_out[...] = jnp.dot(x, wv_ref[...],
                             preferred_element_type=jnp.float32) + bv_ref[...]

    n = feat.shape[0]
    return pl.pallas_call(
        body,
        out_shape=(jax.ShapeDtypeStruct((n, D), jnp.float32),
                   jax.ShapeDtypeStruct((n, D), jnp.float32)),
    )(feat, wpre, bpre, wu, bu, wv, bv)


def _feats(x, wu, bu, wv, bv):
    def body(x_ref, wu_ref, bu_ref, wv_ref, bv_ref, u_out, v_out):
        xv = x_ref[...]
        u_out[...] = jnp.dot(xv, wu_ref[...],
                             preferred_element_type=jnp.float32) + bu_ref[...]
        v_out[...] = jnp.dot(xv, wv_ref[...],
                             preferred_element_type=jnp.float32) + bv_ref[...]

    n = x.shape[0]
    return pl.pallas_call(
        body,
        out_shape=(jax.ShapeDtypeStruct((n, D), jnp.float32),
                   jax.ShapeDtypeStruct((n, D), jnp.float32)),
    )(x, wu, bu, wv, bv)


def _normalize(pos, bp):
    def body(p_ref, bp_ref, o_ref):
        x = p_ref[...] + bp_ref[...]
        ss = jnp.sum(x * x, axis=1, keepdims=True)
        o_ref[...] = x / jnp.maximum(jnp.sqrt(ss), 1e-12)

    return pl.pallas_call(
        body,
        out_shape=jax.ShapeDtypeStruct(pos.shape, jnp.float32),
    )(pos, bp)


def kernel(feat, sp_dist, dists_max, edge_index, anchor_eid,
           W_pre, b_pre, Wu1, bu1, Wv1, bv1, Wp1, bp1,
           Wu2, bu2, Wv2, bv2, Wp2, bp2):
    src = jnp.pad(edge_index[0].astype(jnp.int32), (0, E_PAD - E))
    dst = jnp.pad(edge_index[1].astype(jnp.int32), (0, E_PAD - E))
    sp = jnp.pad(sp_dist[:, 0], (0, E_PAD - E))
    anchor = jnp.pad(anchor_eid.astype(jnp.int32), (0, A_PAD - N * K))

    u1, v1 = _feats_pre(feat, W_pre, b_pre.reshape(1, D),
                        Wu1, bu1.reshape(1, D), Wv1, bv1.reshape(1, D))
    rmsg1 = _edge_messages(u1, v1, src, dst, sp)
    x2 = _anchor_mean(rmsg1, anchor)
    u2, v2 = _feats(x2, Wu2, bu2.reshape(1, D), Wv2, bv2.reshape(1, D))
    p2 = _edge_scores(u2, v2, src, dst, sp, Wp2.reshape(D))
    pos = _scalar_gather(p2, anchor).reshape(N_PAD, K)[:N]
    return _normalize(pos, bp2.reshape(1, 1))
